# Initial kernel scaffold; baseline (speedup 1.0000x reference)
import jax
import jax.numpy as jnp
from jax.experimental import pallas as pl


def kernel(loc_data, conf_data, priors, targets):
    raise NotImplementedError("write your pallas kernel here")


# speedup vs baseline: 3.4055x; 3.4055x over previous
import functools

import jax
import jax.numpy as jnp
from jax.experimental import pallas as pl
from jax.experimental.pallas import tpu as pltpu

_NUM_CLASSES = 21
_OVERLAP_THRESH = 0.5
_TOP_K = 200
_NMS_THRESH = 0.45
_CONF_THRESH = 0.01
_VAR0 = 0.1
_VAR1 = 0.2
_NEG = -1e30
_NWORDS = (_TOP_K + 15) // 16


def _k1(conf_ref, locd_ref, pri_ref, tgt_ref,
        cp_ref, dec_ref, loct_ref, conft_ref):
    P = conf_ref.shape[2]
    conf = conf_ref[0]
    mx = jnp.max(conf, axis=0, keepdims=True)
    e = jnp.exp(conf - mx)
    cp_ref[0] = e / jnp.sum(e, axis=0, keepdims=True)

    pcx = pri_ref[0:1, :]
    pcy = pri_ref[1:2, :]
    pw = pri_ref[2:3, :]
    ph = pri_ref[3:4, :]

    l = locd_ref[0]
    cx = pcx + l[0:1, :] * _VAR0 * pw
    cy = pcy + l[1:2, :] * _VAR0 * ph
    w = pw * jnp.exp(l[2:3, :] * _VAR1)
    h = ph * jnp.exp(l[3:4, :] * _VAR1)
    dx1 = cx - w / 2.0
    dy1 = cy - h / 2.0
    dx2 = w + dx1
    dy2 = h + dy1
    dec_ref[0] = jnp.concatenate([dx1, dy1, dx2, dy2], axis=0)

    pfx1 = pcx - pw / 2.0
    pfy1 = pcy - ph / 2.0
    pfx2 = pcx + pw / 2.0
    pfy2 = pcy + ph / 2.0
    area_p = (pfx2 - pfx1) * (pfy2 - pfy1)

    iota = jax.lax.broadcasted_iota(jnp.int32, (1, P), 1)
    tgt = tgt_ref[0]
    O = tgt.shape[0]

    bto = jnp.full((1, P), -1.0, jnp.float32)
    bti = jnp.zeros((1, P), jnp.int32)
    ious = []
    bpis = []
    for j in range(O):
        tx1 = tgt[j, 0]
        ty1 = tgt[j, 1]
        tx2 = tgt[j, 2]
        ty2 = tgt[j, 3]
        iw = jnp.maximum(jnp.minimum(tx2, pfx2) - jnp.maximum(tx1, pfx1), 0.0)
        ih = jnp.maximum(jnp.minimum(ty2, pfy2) - jnp.maximum(ty1, pfy1), 0.0)
        inter = iw * ih
        area_t = (tx2 - tx1) * (ty2 - ty1)
        iou_j = inter / (area_t + area_p - inter)
        ious.append(iou_j)
        m_j = jnp.max(iou_j)
        bpi_j = jnp.min(jnp.where(iou_j == m_j, iota, jnp.int32(2**30)))
        bpis.append(bpi_j)
        upd = iou_j > bto
        bto = jnp.where(upd, iou_j, bto)
        bti = jnp.where(upd, j, bti)

    for j in range(O):
        mask = iota == bpis[j]
        bto = jnp.where(mask, 2.0, bto)
        bti = jnp.where(mask, j, bti)

    mx1 = jnp.zeros((1, P), jnp.float32)
    my1 = jnp.zeros((1, P), jnp.float32)
    mx2 = jnp.zeros((1, P), jnp.float32)
    my2 = jnp.zeros((1, P), jnp.float32)
    craw = jnp.zeros((1, P), jnp.float32)
    for j in range(O):
        is_j = bti == j
        mx1 = jnp.where(is_j, tgt[j, 0], mx1)
        my1 = jnp.where(is_j, tgt[j, 1], my1)
        mx2 = jnp.where(is_j, tgt[j, 2], mx2)
        my2 = jnp.where(is_j, tgt[j, 3], my2)
        craw = jnp.where(is_j, tgt[j, 4], craw)

    conft = jnp.where(bto < _OVERLAP_THRESH, 0,
                      craw.astype(jnp.int32) + 1)
    conft_ref[0] = conft

    gcx = ((mx1 + mx2) / 2.0 - pcx) / (_VAR0 * pw)
    gcy = ((my1 + my2) / 2.0 - pcy) / (_VAR0 * ph)
    gw = jnp.log((mx2 - mx1) / pw) / _VAR1
    gh = jnp.log((my2 - my1) / ph) / _VAR1
    loct_ref[0] = jnp.concatenate([gcx, gcy, gw, gh], axis=0)


def _k2(cp_ref, dec_ref,
        ka_ref,
        s_scr, rank_scr, wsel_scr, sc_scr, bx_scr, iou_scr):
    P = cp_ref.shape[2]
    C = cp_ref.shape[1]
    NC = C - 1

    cp = cp_ref[0]
    dec = dec_ref[0]
    scores = cp[1:C, :]
    s_scr[...] = jnp.where(scores > _CONF_THRESH, scores, _NEG)
    rank_scr[...] = jnp.full((NC, P), 255, jnp.int32)

    def topk_body(k, carry):
        s = s_scr[...]
        m = jnp.max(s, axis=1, keepdims=True)
        sel = (s == m) & (s > -1e29)
        g = jax.lax.dot_general(
            sel.astype(jnp.float32), dec,
            (((1,), (1,)), ((), ())),
            preferred_element_type=jnp.float32)
        sc_scr[pl.ds(k, 1), :] = jnp.transpose(m)
        bx_scr[pl.ds(k, 1)] = jnp.transpose(g)[None]
        s_scr[...] = jnp.where(sel, _NEG, s)
        rank_scr[...] = jnp.where(sel, k, rank_scr[...])
        return carry

    jax.lax.fori_loop(0, _TOP_K, topk_body, 0)

    sct = sc_scr[...]
    valid = sct > _CONF_THRESH
    x1 = bx_scr[:, 0, :]
    y1 = bx_scr[:, 1, :]
    x2 = bx_scr[:, 2, :]
    y2 = bx_scr[:, 3, :]
    area = (x2 - x1) * (y2 - y1)
    x1t = jnp.transpose(x1)
    y1t = jnp.transpose(y1)
    x2t = jnp.transpose(x2)
    y2t = jnp.transpose(y2)
    areat = jnp.transpose(area)

    CH = 40
    for r0 in range(0, _TOP_K, CH):
        x1c = x1[r0:r0 + CH, :, None]
        y1c = y1[r0:r0 + CH, :, None]
        x2c = x2[r0:r0 + CH, :, None]
        y2c = y2[r0:r0 + CH, :, None]
        xx1 = jnp.maximum(x1c, x1t[None, :, :])
        yy1 = jnp.maximum(y1c, y1t[None, :, :])
        xx2 = jnp.minimum(x2c, x2t[None, :, :])
        yy2 = jnp.minimum(y2c, y2t[None, :, :])
        w = jnp.maximum(xx2 - xx1, 0.0)
        h = jnp.maximum(yy2 - yy1, 0.0)
        inter = w * h
        denom = areat[None, :, :] + area[r0:r0 + CH, :, None] - inter
        iou_scr[r0:r0 + CH] = inter / denom

    validt = jnp.transpose(jnp.where(valid, 1.0, 0.0))
    iota_t = jax.lax.broadcasted_iota(jnp.int32, (1, _TOP_K), 1)

    def nms_body(t, carry):
        supp, keep = carry
        iou_t = iou_scr[pl.ds(t, 1)][0]
        colt = jnp.where(iota_t == t, 1.0, 0.0)
        gt = jnp.where(iota_t > t, 1.0, 0.0)
        niou = jnp.where(iou_t <= _NMS_THRESH, 0.0, 1.0)
        act = jnp.sum(colt * validt * (1.0 - supp),
                      axis=1, keepdims=True)
        keep = keep * (1.0 - colt) + act * colt
        supp = jnp.maximum(supp, act * gt * niou)
        return supp, keep

    supp0 = jnp.zeros((NC, _TOP_K), jnp.float32)
    keep0 = jnp.zeros((NC, _TOP_K), jnp.float32)
    _, keep = jax.lax.fori_loop(0, _TOP_K, nms_body, (supp0, keep0))

    ir = jax.lax.broadcasted_iota(jnp.int32, (_TOP_K, _NWORDS), 0)
    iw = jax.lax.broadcasted_iota(jnp.int32, (_TOP_K, _NWORDS), 1)
    wmat = jnp.where((ir >> 4) == iw,
                     jnp.left_shift(jnp.int32(1), ir & 15),
                     0).astype(jnp.float32)
    words = jnp.dot(keep, wmat,
                    preferred_element_type=jnp.float32)
    wordsi = words.astype(jnp.int32)

    wsel_scr[...] = jnp.zeros((NC, P), jnp.int32)
    for w in range(_NWORDS):
        wsel_scr[...] = jnp.where((rank_scr[...] >> 4) == w,
                                  wordsi[:, w:w + 1], wsel_scr[...])
    bit = jnp.right_shift(wsel_scr[...], rank_scr[...] & 15) & 1
    ka_ref[0] = jnp.where(jnp.sum(bit, axis=0, keepdims=True) > 0,
                          1.0, 0.0)


def _k3(cp_ref, loct_ref, locd_ref, ct_ref, ka_ref, out_ref):
    P = cp_ref.shape[2]
    C = cp_ref.shape[1]
    cp = cp_ref[0]
    ct = ct_ref[0]
    crow = jax.lax.broadcasted_iota(jnp.int32, (C, 1), 0)
    crowf = jnp.where(crow >= 1, 1.0, 0.0)
    cpz = jnp.where(cp > _CONF_THRESH, cp, 0.0) * crowf
    ssum = jnp.sum(cpz, axis=0, keepdims=True)
    efff = jnp.where(ssum != 0.0, 1.0, 0.0)
    mxz = jnp.max(cpz, axis=0, keepdims=True)
    lse = jnp.log(jnp.sum(jnp.exp(cpz - mxz), axis=0, keepdims=True))
    cp_at = jnp.sum(jnp.where(ct == crow, cpz, 0.0), axis=0, keepdims=True)
    picked = cp_at - mxz - lse
    lc_i = -jnp.sum(picked * efff)

    locd = locd_ref[0]
    loct = loct_ref[0]
    locp = locd * ka_ref[0]
    d = locp - loct
    ad = jnp.abs(d)
    sl = jnp.where(ad < 1.0, 0.5 * d * d, ad - 0.5)
    ll_i = jnp.sum(sl * efff)
    n_i = jnp.sum(efff)

    col = jax.lax.broadcasted_iota(jnp.int32, (8, 128), 1)
    out_ref[0] = (jnp.where(col == 0, ll_i, 0.0)
                  + jnp.where(col == 1, lc_i, 0.0)
                  + jnp.where(col == 2, n_i, 0.0))


@functools.partial(jax.jit)
def _run(locd_tm, conf_tm, pri_t, targets):
    B, C, P = conf_tm.shape
    NC = C - 1

    cp, dec, loct, conft = pl.pallas_call(
        _k1,
        grid=(B,),
        in_specs=[
            pl.BlockSpec((1, C, P), lambda b: (b, 0, 0)),
            pl.BlockSpec((1, 4, P), lambda b: (b, 0, 0)),
            pl.BlockSpec((4, P), lambda b: (0, 0)),
            pl.BlockSpec((1, targets.shape[1], 5), lambda b: (b, 0, 0)),
        ],
        out_specs=[
            pl.BlockSpec((1, C, P), lambda b: (b, 0, 0)),
            pl.BlockSpec((1, 4, P), lambda b: (b, 0, 0)),
            pl.BlockSpec((1, 4, P), lambda b: (b, 0, 0)),
            pl.BlockSpec((1, 1, P), lambda b: (b, 0, 0)),
        ],
        out_shape=[
            jax.ShapeDtypeStruct((B, C, P), jnp.float32),
            jax.ShapeDtypeStruct((B, 4, P), jnp.float32),
            jax.ShapeDtypeStruct((B, 4, P), jnp.float32),
            jax.ShapeDtypeStruct((B, 1, P), jnp.int32),
        ],
    )(conf_tm, locd_tm, pri_t, targets)

    ka = pl.pallas_call(
        _k2,
        grid=(B,),
        in_specs=[
            pl.BlockSpec((1, C, P), lambda b: (b, 0, 0)),
            pl.BlockSpec((1, 4, P), lambda b: (b, 0, 0)),
        ],
        out_specs=pl.BlockSpec((1, 1, P), lambda b: (b, 0, 0)),
        out_shape=jax.ShapeDtypeStruct((B, 1, P), jnp.float32),
        scratch_shapes=[
            pltpu.VMEM((NC, P), jnp.float32),
            pltpu.VMEM((NC, P), jnp.int32),
            pltpu.VMEM((NC, P), jnp.int32),
            pltpu.VMEM((_TOP_K, NC), jnp.float32),
            pltpu.VMEM((_TOP_K, 4, NC), jnp.float32),
            pltpu.VMEM((_TOP_K, NC, _TOP_K), jnp.float32),
        ],
    )(cp, dec)

    out = pl.pallas_call(
        _k3,
        grid=(B,),
        in_specs=[
            pl.BlockSpec((1, C, P), lambda b: (b, 0, 0)),
            pl.BlockSpec((1, 4, P), lambda b: (b, 0, 0)),
            pl.BlockSpec((1, 4, P), lambda b: (b, 0, 0)),
            pl.BlockSpec((1, 1, P), lambda b: (b, 0, 0)),
            pl.BlockSpec((1, 1, P), lambda b: (b, 0, 0)),
        ],
        out_specs=pl.BlockSpec((1, 8, 128), lambda b: (b, 0, 0)),
        out_shape=jax.ShapeDtypeStruct((B, 8, 128), jnp.float32),
    )(cp, loct, locd_tm, conft, ka)

    N = jnp.sum(out[:, 0, 2]) * jnp.float32(C)
    return (jnp.sum(out[:, 0, 0]) / N).astype(jnp.float32), \
           (jnp.sum(out[:, 0, 1]) / N).astype(jnp.float32)


def kernel(loc_data, conf_data, priors, targets):
    locd = loc_data.astype(jnp.float32)
    conf = conf_data.astype(jnp.float32)
    pri = priors.astype(jnp.float32)
    tgt = targets.astype(jnp.float32)
    locd_tm = jnp.transpose(locd, (0, 2, 1))
    conf_tm = jnp.transpose(conf, (0, 2, 1))
    pri_t = jnp.transpose(pri)
    return _run(locd_tm, conf_tm, pri_t, tgt)
